# baseline (speedup 1.0000x reference)
import functools

import jax
import jax.numpy as jnp
from jax import lax
from jax.experimental import pallas as pl
from jax.experimental.pallas import tpu as pltpu
from jax.experimental.pallas import tpu_sc as plsc

_VOCAB = 1000000
_C = 200
_D = 64
_B = 1024
_FLAT = _B * _C
_CHUNK = 64
_NC, _NS = 2, 16
_NW = _NC * _NS
_CPW = _FLAT // (_CHUNK * _NW)
_IPW = _CPW * _CHUNK
_NBUF = 5
_BLKS = _CPW // _NBUF
_LANES = 16
_FD = 2 * _D


def _sc_gather_add(text_flat, table2, pos):
    mesh = plsc.VectorSubcoreMesh(core_axis_name="c", subcore_axis_name="s",
                                  num_cores=_NC, num_subcores=_NS)

    @functools.partial(
        pl.kernel,
        out_type=jax.ShapeDtypeStruct((_FLAT, _D), jnp.float32),
        mesh=mesh,
        scratch_types=[
            pltpu.VMEM((_IPW,), jnp.int32),
            pltpu.VMEM((_IPW,), jnp.int32),
            pltpu.VMEM((_C, _D), jnp.float32),
            pltpu.VMEM((_NBUF, _CHUNK, _FD), jnp.float32),
            pltpu.VMEM((_NBUF, _CHUNK, _D), jnp.float32),
            pltpu.SemaphoreType.DMA((_NBUF,)),
            pltpu.SemaphoreType.DMA((_NBUF,)),
        ],
        compiler_params=pltpu.CompilerParams(use_tc_tiling_on_sc=True,
                                             needs_layout_passes=False),
    )
    def k(text_hbm, table_hbm, pos_hbm, out_hbm,
          raw_v, idx_v, pos_v, fbuf_v, obuf_v, gsem, ssem):
        wid = lax.axis_index("s") * _NC + lax.axis_index("c")
        pltpu.sync_copy(pos_hbm, pos_v)
        base = wid * _IPW
        pltpu.sync_copy(text_hbm.at[pl.ds(base, _IPW)], raw_v)

        def halve(m, _):
            sl = pl.ds(m * _LANES, _LANES)
            idx_v[sl] = lax.shift_right_logical(raw_v[sl], 1)
            return ()

        lax.fori_loop(0, _IPW // _LANES, halve, (), unroll=8)
        chunk0 = wid * _CPW

        def gather_start(j, b):
            ioff = pl.multiple_of(j * _CHUNK, _CHUNK)
            pltpu.async_copy(table_hbm.at[idx_v.at[pl.ds(ioff, _CHUNK)]],
                             fbuf_v.at[b], gsem.at[b])

        def gather_wait(b):
            pltpu.make_async_copy(table_hbm.at[idx_v.at[pl.ds(0, _CHUNK)]],
                                  fbuf_v.at[b], gsem.at[b]).wait()

        def store_start(j, b):
            off = pl.multiple_of((chunk0 + j) * _CHUNK, _CHUNK)
            pltpu.async_copy(obuf_v.at[b], out_hbm.at[pl.ds(off, _CHUNK)],
                             ssem.at[b])

        def store_wait(b):
            pltpu.make_async_copy(obuf_v.at[b], out_hbm.at[pl.ds(0, _CHUNK)],
                                  ssem.at[b]).wait()

        def step(j, b, first_block, last_block):
            gather_wait(b)
            if not first_block:
                store_wait(b)
            ioff = pl.multiple_of(j * _CHUNK, _CHUNK)
            p0 = lax.rem(j * _CHUNK, _C)
            lanes = lax.broadcasted_iota(jnp.int32, (_LANES,), 0)
            fb = fbuf_v.at[b]
            ob = obuf_v.at[b]
            for m in range(_CHUNK // _LANES):
                rvec = lanes + (m * _LANES)
                hvec = (raw_v[pl.ds(ioff + m * _LANES, _LANES)] & 1) * _D
                q = lanes + (p0 + m * _LANES)
                pvec = q - jnp.where(q >= _C, _C, 0)

                def col_body(c, _):
                    cvec = jnp.full((_LANES,), c, jnp.int32)
                    vals = plsc.load_gather(fb, [rvec, hvec + c])
                    posv = plsc.load_gather(pos_v, [pvec, cvec])
                    plsc.store_scatter(ob, [rvec, cvec], vals + posv)
                    return ()

                lax.fori_loop(0, _D, col_body, (), unroll=4)
            if not last_block:
                gather_start(j + _NBUF, b)
            store_start(j, b)

        for b in range(_NBUF):
            gather_start(b, b)
        for b in range(_NBUF):
            step(b, b, True, False)

        def mid_block(jo, _):
            for b in range(_NBUF):
                step(jo * _NBUF + b, b, False, False)
            return ()

        lax.fori_loop(1, _BLKS - 1, mid_block, ())
        for b in range(_NBUF):
            step((_BLKS - 1) * _NBUF + b, b, False, True)
        for b in range(_NBUF):
            store_wait(b)

    return k(text_flat, table2, pos)


def _mask_body(o_ref):
    i = lax.broadcasted_iota(jnp.int32, (_C, _C), 0)
    j = lax.broadcasted_iota(jnp.int32, (_C, _C), 1)
    o_ref[...] = jnp.where(j > i, -jnp.inf, 0.0).astype(jnp.float32)


def _causal_mask():
    return pl.pallas_call(
        _mask_body,
        out_shape=jax.ShapeDtypeStruct((_C, _C), jnp.float32),
    )()


def kernel(text, token_embedding, positional_embedding):
    text_flat = text.astype(jnp.int32).reshape(_FLAT)
    table2 = token_embedding.astype(jnp.float32).reshape(_VOCAB // 2, _FD)
    x = _sc_gather_add(text_flat, table2,
                       positional_embedding.astype(jnp.float32))
    return (x.reshape(_B, _C, _D), _causal_mask())


# speedup vs baseline: 1.7287x; 1.7287x over previous
import functools

import jax
import jax.numpy as jnp
from jax import lax
from jax.experimental import pallas as pl
from jax.experimental.pallas import tpu as pltpu
from jax.experimental.pallas import tpu_sc as plsc

_VOCAB = 1000000
_C = 200
_D = 64
_B = 1024
_FLAT = _B * _C
_CHUNK = 64
_NC, _NS = 2, 16
_NW = _NC * _NS
_CPW = _FLAT // (_CHUNK * _NW)
_IPW = _CPW * _CHUNK
_NBUF = 5
_BLKS = _CPW // _NBUF
_LANES = 16
_FD = 2 * _D


def _sc_gather_add(text_flat, table2, pos):
    mesh = plsc.VectorSubcoreMesh(core_axis_name="c", subcore_axis_name="s",
                                  num_cores=_NC, num_subcores=_NS)

    @functools.partial(
        pl.kernel,
        out_type=jax.ShapeDtypeStruct((_FLAT, _D), jnp.float32),
        mesh=mesh,
        scratch_types=[
            pltpu.VMEM((_IPW,), jnp.int32),
            pltpu.VMEM((_IPW,), jnp.int32),
            pltpu.VMEM((_C, _D), jnp.float32),
            pltpu.VMEM((_NBUF, _CHUNK, _FD), jnp.float32),
            pltpu.VMEM((_NBUF, _CHUNK, _D), jnp.float32),
            pltpu.SemaphoreType.DMA((_NBUF,)),
            pltpu.SemaphoreType.DMA((_NBUF,)),
        ],
        compiler_params=pltpu.CompilerParams(use_tc_tiling_on_sc=True,
                                             needs_layout_passes=False),
    )
    def k(text_hbm, table_hbm, pos_hbm, out_hbm,
          raw_v, idx_v, pos_v, fbuf_v, obuf_v, gsem, ssem):
        wid = lax.axis_index("s") * _NC + lax.axis_index("c")
        pltpu.sync_copy(pos_hbm, pos_v)
        base = wid * _IPW
        pltpu.sync_copy(text_hbm.at[pl.ds(base, _IPW)], raw_v)

        def halve(m, _):
            sl = pl.ds(m * _LANES, _LANES)
            idx_v[sl] = lax.shift_right_logical(raw_v[sl], 1)
            return ()

        lax.fori_loop(0, _IPW // _LANES, halve, (), unroll=8)
        chunk0 = wid * _CPW

        def gather_start(j, b):
            ioff = pl.multiple_of(j * _CHUNK, _CHUNK)
            pltpu.async_copy(table_hbm.at[idx_v.at[pl.ds(ioff, _CHUNK)]],
                             fbuf_v.at[b], gsem.at[b])

        def gather_wait(b):
            pltpu.make_async_copy(table_hbm.at[idx_v.at[pl.ds(0, _CHUNK)]],
                                  fbuf_v.at[b], gsem.at[b]).wait()

        def store_start(j, b):
            off = pl.multiple_of((chunk0 + j) * _CHUNK, _CHUNK)
            pltpu.async_copy(obuf_v.at[b], out_hbm.at[pl.ds(off, _CHUNK)],
                             ssem.at[b])

        def store_wait(b):
            pltpu.make_async_copy(obuf_v.at[b], out_hbm.at[pl.ds(0, _CHUNK)],
                                  ssem.at[b]).wait()

        def step(j, b, first_block, last_block):
            gather_wait(b)
            if not first_block:
                store_wait(b)
            ioff = pl.multiple_of(j * _CHUNK, _CHUNK)
            p0 = lax.rem(j * _CHUNK, _C)

            def row_body(r, p):
                rsplat = jnp.full((_LANES,), ioff + r, jnp.int32)
                msk = (plsc.load_gather(raw_v, [rsplat]) & 1) == 1
                for kk in range(_D // _LANES):
                    sl = pl.ds(kk * _LANES, _LANES)
                    lo = fbuf_v[b, r, sl]
                    hi = fbuf_v[b, r, pl.ds(_D + kk * _LANES, _LANES)]
                    obuf_v[b, r, sl] = jnp.where(msk, hi, lo) + pos_v[p, sl]
                p = p + 1
                return lax.select(p == _C, 0, p)

            lax.fori_loop(0, _CHUNK, row_body, p0, unroll=2)
            if not last_block:
                gather_start(j + _NBUF, b)
            store_start(j, b)

        for b in range(_NBUF):
            gather_start(b, b)
        for b in range(_NBUF):
            step(b, b, True, False)

        def mid_block(jo, _):
            for b in range(_NBUF):
                step(jo * _NBUF + b, b, False, False)
            return ()

        lax.fori_loop(1, _BLKS - 1, mid_block, ())
        for b in range(_NBUF):
            step((_BLKS - 1) * _NBUF + b, b, False, True)
        for b in range(_NBUF):
            store_wait(b)

    return k(text_flat, table2, pos)


def _mask_body(o_ref):
    i = lax.broadcasted_iota(jnp.int32, (_C, _C), 0)
    j = lax.broadcasted_iota(jnp.int32, (_C, _C), 1)
    o_ref[...] = jnp.where(j > i, -jnp.inf, 0.0).astype(jnp.float32)


def _causal_mask():
    return pl.pallas_call(
        _mask_body,
        out_shape=jax.ShapeDtypeStruct((_C, _C), jnp.float32),
    )()


def kernel(text, token_embedding, positional_embedding):
    text_flat = text.astype(jnp.int32).reshape(_FLAT)
    table2 = token_embedding.astype(jnp.float32).reshape(_VOCAB // 2, _FD)
    x = _sc_gather_add(text_flat, table2,
                       positional_embedding.astype(jnp.float32))
    return (x.reshape(_B, _C, _D), _causal_mask())
